# Initial kernel scaffold; baseline (speedup 1.0000x reference)
import jax
import jax.numpy as jnp
from jax.experimental import pallas as pl


def kernel(x, edge_index, W_self0, W_neigh0, b0, W_self1, W_neigh1, b1):
    raise NotImplementedError("write your pallas kernel here")


# speedup vs baseline: 3.8805x; 3.8805x over previous
import functools

import jax
import jax.numpy as jnp
from jax import lax
from jax.experimental import pallas as pl
from jax.experimental.pallas import tpu as pltpu
from jax.experimental.pallas import tpu_sc as plsc

N = 10000
E = 320000
D = 128
NP = 10240
NC = 2
NS = 16
HALF = NP // NC
TRASH = HALF
ACC_ROWS = HALF + 8
STRIPE = HALF // NS
CHUNK = 80
NCHUNK = (E // NS) // CHUNK
L = 16


def _sc_body(with_deg, x_hbm, src_hbm, dst_hbm, z2_hbm,
             agg_out, deg_out, src_v, dst_v, sidx_v, rows_v, ones_v,
             degtmp_v, acc_sh, deg_sh):
    c = lax.axis_index("c")
    s = lax.axis_index("s")
    r0 = s * STRIPE
    base = c * HALF

    pltpu.sync_copy(z2_hbm, acc_sh.at[pl.ds(r0, STRIPE)])
    if with_deg:
        for i in range(STRIPE // L):
            degtmp_v[0, pl.ds(i * L, L)] = jnp.zeros((L,), jnp.float32)
        pltpu.sync_copy(degtmp_v.at[0], deg_sh.at[pl.ds(r0, STRIPE)])
        for i in range(CHUNK // L):
            ones_v[pl.ds(i * L, L)] = jnp.ones((L,), jnp.float32)

    pltpu.sync_copy(src_hbm.at[s], src_v)
    pltpu.sync_copy(dst_hbm.at[s], dst_v)
    plsc.subcore_barrier()

    def step(ci, carry):
        for k in range(CHUNK // L):
            dv = dst_v[ci, pl.ds(k * L, L)]
            t = dv - base
            valid = (t >= 0) & (t < HALF)
            sidx_v[pl.ds(k * L, L)] = jnp.where(valid, t, TRASH)
        pltpu.sync_copy(x_hbm.at[src_v.at[ci]], rows_v)
        pltpu.sync_copy(rows_v, acc_sh.at[sidx_v], add=True)
        if with_deg:
            pltpu.sync_copy(ones_v, deg_sh.at[sidx_v], add=True)
        return carry

    lax.fori_loop(0, NCHUNK, step, 0)
    plsc.subcore_barrier()

    pltpu.sync_copy(acc_sh.at[pl.ds(r0, STRIPE)],
                    agg_out.at[pl.ds(base + r0, STRIPE)])
    if with_deg:
        pltpu.sync_copy(deg_sh.at[pl.ds(r0, STRIPE)], degtmp_v.at[0])
        pltpu.sync_copy(degtmp_v, deg_out.at[c * NS + s])


def _make_sc_agg(with_deg):
    mesh = plsc.VectorSubcoreMesh(core_axis_name="c", subcore_axis_name="s")
    out_type = [jax.ShapeDtypeStruct((NP, D), jnp.float32)]
    if with_deg:
        out_type.append(jax.ShapeDtypeStruct((NC * NS, 1, STRIPE), jnp.float32))
    scratch = [
        pltpu.VMEM((NCHUNK, CHUNK), jnp.int32),
        pltpu.VMEM((NCHUNK, CHUNK), jnp.int32),
        pltpu.VMEM((CHUNK,), jnp.int32),
        pltpu.VMEM((CHUNK, D), jnp.float32),
        pltpu.VMEM((CHUNK,), jnp.float32),
        pltpu.VMEM((1, STRIPE), jnp.float32),
        pltpu.VMEM_SHARED((ACC_ROWS, D), jnp.float32),
        pltpu.VMEM_SHARED((ACC_ROWS,), jnp.float32),
    ]

    def body(x_hbm, src_hbm, dst_hbm, z2_hbm, *rest):
        if with_deg:
            agg_out, deg_out = rest[0], rest[1]
            rest = rest[2:]
        else:
            agg_out, deg_out = rest[0], None
            rest = rest[1:]
        _sc_body(with_deg, x_hbm, src_hbm, dst_hbm, z2_hbm,
                 agg_out, deg_out, *rest)

    return pl.kernel(body, out_type=tuple(out_type), mesh=mesh,
                     scratch_types=scratch)


_sc_agg_deg = _make_sc_agg(True)
_sc_agg = _make_sc_agg(False)


def _tc_layer_body(relu, h_ref, a_ref, d_ref, ws_ref, wn_ref, b_ref, o_ref):
    hv = h_ref[...]
    inv = 1.0 / jnp.maximum(d_ref[...], 1.0)
    hn = a_ref[...] * inv
    out = (jnp.dot(hv, ws_ref[...], preferred_element_type=jnp.float32,
                   precision=lax.Precision.HIGHEST)
           + jnp.dot(hn, wn_ref[...], preferred_element_type=jnp.float32,
                     precision=lax.Precision.HIGHEST)
           + b_ref[...])
    if relu:
        out = jnp.maximum(out, 0.0)
    o_ref[...] = out


def _tc_layer(h, agg, degc, W_self, W_neigh, b, relu):
    R = 1280
    grid = NP // R
    return pl.pallas_call(
        functools.partial(_tc_layer_body, relu),
        grid=(grid,),
        in_specs=[
            pl.BlockSpec((R, D), lambda i: (i, 0)),
            pl.BlockSpec((R, D), lambda i: (i, 0)),
            pl.BlockSpec((R, 1), lambda i: (i, 0)),
            pl.BlockSpec((D, D), lambda i: (0, 0)),
            pl.BlockSpec((D, D), lambda i: (0, 0)),
            pl.BlockSpec((1, D), lambda i: (0, 0)),
        ],
        out_specs=pl.BlockSpec((R, D), lambda i: (i, 0)),
        out_shape=jax.ShapeDtypeStruct((NP, D), jnp.float32),
    )(h, agg, degc, W_self, W_neigh, b)


def kernel(x, edge_index, W_self0, W_neigh0, b0, W_self1, W_neigh1, b1):
    x_pad = jnp.pad(x, ((0, NP - N), (0, 0)))
    src = edge_index[0].reshape(NS, NCHUNK, CHUNK)
    dst = edge_index[1].reshape(NS, NCHUNK, CHUNK)
    z2 = jnp.zeros((STRIPE, D), jnp.float32)

    agg0, deg = _sc_agg_deg(x_pad, src, dst, z2)
    degc = deg.reshape(NP, 1)
    h1 = _tc_layer(x_pad, agg0, degc, W_self0, W_neigh0,
                   b0.reshape(1, D), relu=True)
    (agg1,) = _sc_agg(h1, src, dst, z2)
    out = _tc_layer(h1, agg1, degc, W_self1, W_neigh1,
                    b1.reshape(1, D), relu=False)
    return out[:N]


# speedup vs baseline: 4.9589x; 1.2779x over previous
import functools

import jax
import jax.numpy as jnp
from jax import lax
from jax.experimental import pallas as pl
from jax.experimental.pallas import tpu as pltpu
from jax.experimental.pallas import tpu_sc as plsc

N = 10000
E = 320000
D = 128
NP = 10240
NC = 2
NS = 16
HALF = NP // NC
TRASH = HALF
ACC_ROWS = HALF + 8
STRIPE = HALF // NS
CHUNK = 80
NCHUNK = (E // NS) // CHUNK
L = 16


def _sc_body(with_deg, x_hbm, src_hbm, dst_hbm, z2_hbm,
             agg_out, deg_out, src_v, dst_v, sidx_v, rows_v, ones_v,
             degtmp_v, acc_sh, deg_sh, gsem0, gsem1):
    c = lax.axis_index("c")
    s = lax.axis_index("s")
    r0 = s * STRIPE
    base = c * HALF

    pltpu.sync_copy(z2_hbm, acc_sh.at[pl.ds(r0, STRIPE)])
    if with_deg:
        for i in range(STRIPE // L):
            degtmp_v[0, pl.ds(i * L, L)] = jnp.zeros((L,), jnp.float32)
        pltpu.sync_copy(degtmp_v.at[0], deg_sh.at[pl.ds(r0, STRIPE)])
        for i in range(CHUNK // L):
            ones_v[pl.ds(i * L, L)] = jnp.ones((L,), jnp.float32)

    pltpu.sync_copy(src_hbm.at[s], src_v)
    pltpu.sync_copy(dst_hbm.at[s], dst_v)
    plsc.subcore_barrier()

    gsems = (gsem0, gsem1)
    pltpu.async_copy(x_hbm.at[src_v.at[0]], rows_v.at[0], gsem0)

    def outer(g, carry):
        for b in range(2):
            ci = 2 * g + b

            @pl.when(ci + 1 < NCHUNK)
            def _():
                pltpu.async_copy(x_hbm.at[src_v.at[ci + 1]],
                                 rows_v.at[1 - b], gsems[1 - b])

            for k in range(CHUNK // L):
                dv = dst_v[ci, pl.ds(k * L, L)]
                t = dv - base
                valid = (t >= 0) & (t < HALF)
                sidx_v[pl.ds(k * L, L)] = jnp.where(valid, t, TRASH)
            pltpu.make_async_copy(x_hbm.at[pl.ds(0, CHUNK)],
                                  rows_v.at[b], gsems[b]).wait()
            pltpu.sync_copy(rows_v.at[b], acc_sh.at[sidx_v], add=True)
            if with_deg:
                pltpu.sync_copy(ones_v, deg_sh.at[sidx_v], add=True)
        return carry

    lax.fori_loop(0, NCHUNK // 2, outer, 0)
    plsc.subcore_barrier()

    pltpu.sync_copy(acc_sh.at[pl.ds(r0, STRIPE)],
                    agg_out.at[pl.ds(base + r0, STRIPE)])
    if with_deg:
        pltpu.sync_copy(deg_sh.at[pl.ds(r0, STRIPE)], degtmp_v.at[0])
        pltpu.sync_copy(degtmp_v, deg_out.at[c * NS + s])


def _make_sc_agg(with_deg):
    mesh = plsc.VectorSubcoreMesh(core_axis_name="c", subcore_axis_name="s")
    out_type = [jax.ShapeDtypeStruct((NP, D), jnp.float32)]
    if with_deg:
        out_type.append(jax.ShapeDtypeStruct((NC * NS, 1, STRIPE), jnp.float32))
    scratch = [
        pltpu.VMEM((NCHUNK, CHUNK), jnp.int32),
        pltpu.VMEM((NCHUNK, CHUNK), jnp.int32),
        pltpu.VMEM((CHUNK,), jnp.int32),
        pltpu.VMEM((2, CHUNK, D), jnp.float32),
        pltpu.VMEM((CHUNK,), jnp.float32),
        pltpu.VMEM((1, STRIPE), jnp.float32),
        pltpu.VMEM_SHARED((ACC_ROWS, D), jnp.float32),
        pltpu.VMEM_SHARED((ACC_ROWS,), jnp.float32),
        pltpu.SemaphoreType.DMA,
        pltpu.SemaphoreType.DMA,
    ]

    def body(x_hbm, src_hbm, dst_hbm, z2_hbm, *rest):
        if with_deg:
            agg_out, deg_out = rest[0], rest[1]
            rest = rest[2:]
        else:
            agg_out, deg_out = rest[0], None
            rest = rest[1:]
        _sc_body(with_deg, x_hbm, src_hbm, dst_hbm, z2_hbm,
                 agg_out, deg_out, *rest)

    return pl.kernel(body, out_type=tuple(out_type), mesh=mesh,
                     scratch_types=scratch)


_sc_agg_deg = _make_sc_agg(True)
_sc_agg = _make_sc_agg(False)


def _tc_layer_body(relu, h_ref, a_ref, d_ref, ws_ref, wn_ref, b_ref, o_ref):
    hv = h_ref[...]
    inv = 1.0 / jnp.maximum(d_ref[...], 1.0)
    hn = a_ref[...] * inv
    out = (jnp.dot(hv, ws_ref[...], preferred_element_type=jnp.float32,
                   precision=lax.Precision.HIGHEST)
           + jnp.dot(hn, wn_ref[...], preferred_element_type=jnp.float32,
                     precision=lax.Precision.HIGHEST)
           + b_ref[...])
    if relu:
        out = jnp.maximum(out, 0.0)
    o_ref[...] = out


def _tc_layer(h, agg, degc, W_self, W_neigh, b, relu):
    R = 1280
    grid = NP // R
    return pl.pallas_call(
        functools.partial(_tc_layer_body, relu),
        grid=(grid,),
        in_specs=[
            pl.BlockSpec((R, D), lambda i: (i, 0)),
            pl.BlockSpec((R, D), lambda i: (i, 0)),
            pl.BlockSpec((R, 1), lambda i: (i, 0)),
            pl.BlockSpec((D, D), lambda i: (0, 0)),
            pl.BlockSpec((D, D), lambda i: (0, 0)),
            pl.BlockSpec((1, D), lambda i: (0, 0)),
        ],
        out_specs=pl.BlockSpec((R, D), lambda i: (i, 0)),
        out_shape=jax.ShapeDtypeStruct((NP, D), jnp.float32),
    )(h, agg, degc, W_self, W_neigh, b)


def kernel(x, edge_index, W_self0, W_neigh0, b0, W_self1, W_neigh1, b1):
    x_pad = jnp.pad(x, ((0, NP - N), (0, 0)))
    src = edge_index[0].reshape(NS, NCHUNK, CHUNK)
    dst = edge_index[1].reshape(NS, NCHUNK, CHUNK)
    z2 = jnp.zeros((STRIPE, D), jnp.float32)

    agg0, deg = _sc_agg_deg(x_pad, src, dst, z2)
    degc = deg.reshape(NP, 1)
    h1 = _tc_layer(x_pad, agg0, degc, W_self0, W_neigh0,
                   b0.reshape(1, D), relu=True)
    (agg1,) = _sc_agg(h1, src, dst, z2)
    out = _tc_layer(h1, agg1, degc, W_self1, W_neigh1,
                    b1.reshape(1, D), relu=False)
    return out[:N]


# speedup vs baseline: 4.9602x; 1.0003x over previous
import functools

import jax
import jax.numpy as jnp
from jax import lax
from jax.experimental import pallas as pl
from jax.experimental.pallas import tpu as pltpu
from jax.experimental.pallas import tpu_sc as plsc

N = 10000
E = 320000
D = 128
NP = 10240
NC = 2
NS = 16
HALF = NP // NC
TRASH = HALF
ACC_ROWS = HALF + 8
STRIPE = HALF // NS
CHUNK = 80
NCHUNK = (E // NS) // CHUNK
L = 16


def _sc_body(with_deg, x_hbm, src_hbm, dst_hbm, z2_hbm,
             agg_out, deg_out, src_v, dst_v, sidx_v, rows_v, ones_v,
             degtmp_v, acc_sh, deg_sh, gsem0, gsem1, ssem0, ssem1,
             dsem0, dsem1):
    c = lax.axis_index("c")
    s = lax.axis_index("s")
    r0 = s * STRIPE
    base = c * HALF

    pltpu.sync_copy(z2_hbm, acc_sh.at[pl.ds(r0, STRIPE)])
    if with_deg:
        for i in range(STRIPE // L):
            degtmp_v[0, pl.ds(i * L, L)] = jnp.zeros((L,), jnp.float32)
        pltpu.sync_copy(degtmp_v.at[0], deg_sh.at[pl.ds(r0, STRIPE)])
        for i in range(CHUNK // L):
            ones_v[pl.ds(i * L, L)] = jnp.ones((L,), jnp.float32)

    pltpu.sync_copy(src_hbm.at[s], src_v)
    pltpu.sync_copy(dst_hbm.at[s], dst_v)
    plsc.subcore_barrier()

    gsems = (gsem0, gsem1)
    ssems = (ssem0, ssem1)
    dsems = (dsem0, dsem1)

    def _wait_gather(b, ci):
        pltpu.make_async_copy(x_hbm.at[src_v.at[ci]],
                              rows_v.at[b], gsems[b]).wait()

    def _wait_scatter(b):
        pltpu.make_async_copy(rows_v.at[b],
                              acc_sh.at[sidx_v.at[b]], ssems[b]).wait()
        if with_deg:
            pltpu.make_async_copy(ones_v,
                                  deg_sh.at[sidx_v.at[b]], dsems[b]).wait()

    pltpu.async_copy(x_hbm.at[src_v.at[0]], rows_v.at[0], gsem0)

    def outer(g, carry):
        for b in range(2):
            ci = 2 * g + b

            @pl.when(ci > 0)
            def _():
                _wait_scatter(1 - b)

            @pl.when(ci + 1 < NCHUNK)
            def _():
                pltpu.async_copy(x_hbm.at[src_v.at[ci + 1]],
                                 rows_v.at[1 - b], gsems[1 - b])

            for k in range(CHUNK // L):
                dv = dst_v[ci, pl.ds(k * L, L)]
                t = dv - base
                valid = (t >= 0) & (t < HALF)
                sidx_v[b, pl.ds(k * L, L)] = jnp.where(valid, t, TRASH)
            _wait_gather(b, ci)
            pltpu.async_copy(rows_v.at[b], acc_sh.at[sidx_v.at[b]],
                             ssems[b], add=True)
            if with_deg:
                pltpu.async_copy(ones_v, deg_sh.at[sidx_v.at[b]],
                                 dsems[b], add=True)
        return carry

    lax.fori_loop(0, NCHUNK // 2, outer, 0)
    _wait_scatter((NCHUNK - 1) % 2)
    plsc.subcore_barrier()

    pltpu.sync_copy(acc_sh.at[pl.ds(r0, STRIPE)],
                    agg_out.at[pl.ds(base + r0, STRIPE)])
    if with_deg:
        pltpu.sync_copy(deg_sh.at[pl.ds(r0, STRIPE)], degtmp_v.at[0])
        pltpu.sync_copy(degtmp_v, deg_out.at[c * NS + s])


def _make_sc_agg(with_deg):
    mesh = plsc.VectorSubcoreMesh(core_axis_name="c", subcore_axis_name="s")
    out_type = [jax.ShapeDtypeStruct((NP, D), jnp.float32)]
    if with_deg:
        out_type.append(jax.ShapeDtypeStruct((NC * NS, 1, STRIPE), jnp.float32))
    scratch = [
        pltpu.VMEM((NCHUNK, CHUNK), jnp.int32),
        pltpu.VMEM((NCHUNK, CHUNK), jnp.int32),
        pltpu.VMEM((2, CHUNK), jnp.int32),
        pltpu.VMEM((2, CHUNK, D), jnp.float32),
        pltpu.VMEM((CHUNK,), jnp.float32),
        pltpu.VMEM((1, STRIPE), jnp.float32),
        pltpu.VMEM_SHARED((ACC_ROWS, D), jnp.float32),
        pltpu.VMEM_SHARED((ACC_ROWS,), jnp.float32),
        pltpu.SemaphoreType.DMA,
        pltpu.SemaphoreType.DMA,
        pltpu.SemaphoreType.DMA,
        pltpu.SemaphoreType.DMA,
        pltpu.SemaphoreType.DMA,
        pltpu.SemaphoreType.DMA,
    ]

    def body(x_hbm, src_hbm, dst_hbm, z2_hbm, *rest):
        if with_deg:
            agg_out, deg_out = rest[0], rest[1]
            rest = rest[2:]
        else:
            agg_out, deg_out = rest[0], None
            rest = rest[1:]
        _sc_body(with_deg, x_hbm, src_hbm, dst_hbm, z2_hbm,
                 agg_out, deg_out, *rest)

    return pl.kernel(body, out_type=tuple(out_type), mesh=mesh,
                     scratch_types=scratch)


_sc_agg_deg = _make_sc_agg(True)
_sc_agg = _make_sc_agg(False)


def _tc_layer_body(relu, h_ref, a_ref, d_ref, ws_ref, wn_ref, b_ref, o_ref):
    hv = h_ref[...]
    inv = 1.0 / jnp.maximum(d_ref[...], 1.0)
    hn = a_ref[...] * inv
    out = (jnp.dot(hv, ws_ref[...], preferred_element_type=jnp.float32,
                   precision=lax.Precision.HIGHEST)
           + jnp.dot(hn, wn_ref[...], preferred_element_type=jnp.float32,
                     precision=lax.Precision.HIGHEST)
           + b_ref[...])
    if relu:
        out = jnp.maximum(out, 0.0)
    o_ref[...] = out


def _tc_layer(h, agg, degc, W_self, W_neigh, b, relu):
    R = 1280
    grid = NP // R
    return pl.pallas_call(
        functools.partial(_tc_layer_body, relu),
        grid=(grid,),
        in_specs=[
            pl.BlockSpec((R, D), lambda i: (i, 0)),
            pl.BlockSpec((R, D), lambda i: (i, 0)),
            pl.BlockSpec((R, 1), lambda i: (i, 0)),
            pl.BlockSpec((D, D), lambda i: (0, 0)),
            pl.BlockSpec((D, D), lambda i: (0, 0)),
            pl.BlockSpec((1, D), lambda i: (0, 0)),
        ],
        out_specs=pl.BlockSpec((R, D), lambda i: (i, 0)),
        out_shape=jax.ShapeDtypeStruct((NP, D), jnp.float32),
    )(h, agg, degc, W_self, W_neigh, b)


def kernel(x, edge_index, W_self0, W_neigh0, b0, W_self1, W_neigh1, b1):
    x_pad = jnp.pad(x, ((0, NP - N), (0, 0)))
    src = edge_index[0].reshape(NS, NCHUNK, CHUNK)
    dst = edge_index[1].reshape(NS, NCHUNK, CHUNK)
    z2 = jnp.zeros((STRIPE, D), jnp.float32)

    agg0, deg = _sc_agg_deg(x_pad, src, dst, z2)
    degc = deg.reshape(NP, 1)
    h1 = _tc_layer(x_pad, agg0, degc, W_self0, W_neigh0,
                   b0.reshape(1, D), relu=True)
    (agg1,) = _sc_agg(h1, src, dst, z2)
    out = _tc_layer(h1, agg1, degc, W_self1, W_neigh1,
                    b1.reshape(1, D), relu=False)
    return out[:N]
